# baseline (speedup 1.0000x reference)
import functools

import jax
import jax.numpy as jnp
from jax import lax
from jax.experimental import pallas as pl
from jax.experimental.pallas import tpu as pltpu
from jax.experimental.pallas import tpu_sc as plsc

N = 4096
SEQ = 2048
HID = 1024
ENC = 512
E = 8
TILE = 512
TLE = 512
NT2 = N // TLE
NITEMS = NT2 + E - 1
NSEG = 32
NW = 32
RPW = N // NW
GCH = 32

_SC_KERNELS = None


def _enc_route_body(ids_ref, x_ref, w1_ref, b1_ref, w2_ref, b2_ref,
                    h_ref, pos_ref, st_ref, se_ref, slo_ref, shi_ref):
    t = pl.program_id(0)
    f32 = jnp.float32
    i32 = jnp.int32

    f = x_ref[...]
    h1 = jnp.maximum(
        jnp.dot(f, w1_ref[...], preferred_element_type=f32) + b1_ref[...], 0.0)
    h1b = h1.astype(jnp.bfloat16)
    h_ref[...] = jnp.maximum(
        jnp.dot(h1b, w2_ref[...], preferred_element_type=f32) + b2_ref[...], 0.0)

    ids_full = ids_ref[...]
    onehot = (ids_full == lax.broadcasted_iota(i32, (N, E), 1)).astype(f32)
    counts = jnp.sum(onehot, axis=0, keepdims=True)
    tri = (lax.broadcasted_iota(i32, (E, E), 0)
           < lax.broadcasted_iota(i32, (E, E), 1)).astype(f32)
    offs = jnp.dot(counts, tri, preferred_element_type=f32, precision=lax.Precision.HIGHEST)

    r0 = t * TILE
    ids_tile = ids_ref[pl.ds(r0, TILE), :]
    onehot_t = (ids_tile == lax.broadcasted_iota(i32, (TILE, E), 1)).astype(f32)
    base_mask = (lax.broadcasted_iota(i32, (1, N), 1) < r0).astype(f32)
    base = jnp.dot(base_mask, onehot, preferred_element_type=f32)
    tril = (lax.broadcasted_iota(i32, (TILE, TILE), 1)
            <= lax.broadcasted_iota(i32, (TILE, TILE), 0)).astype(f32)
    cum = base + jnp.dot(tril, onehot_t, preferred_element_type=f32)
    posf = jnp.sum(onehot_t * (offs + cum - 1.0), axis=1, keepdims=True)
    pos_ref[...] = posf.astype(i32)

    lane = lax.broadcasted_iota(i32, (1, NSEG), 1)
    prow = lax.broadcasted_iota(i32, (E, NSEG), 0)
    pcol = lax.broadcasted_iota(i32, (E, NSEG), 1)
    P = ((pcol == prow + (NT2 - 1)) & (prow >= 1)).astype(f32)
    u_off = jnp.dot(offs, P, preferred_element_type=f32, precision=lax.Precision.HIGHEST)
    u = jnp.where(lane < NT2, lane.astype(f32) * TLE,
                  jnp.where(lane < NITEMS, u_off, float(N)))

    ii16 = lax.broadcasted_iota(i32, (NSEG, NSEG), 0)
    jj16 = lax.broadcasted_iota(i32, (NSEG, NSEG), 1)
    ur = jnp.broadcast_to(u, (NSEG, NSEG))
    D = jnp.where(ii16 == jj16, ur, 0.0)
    uc = jnp.dot(D, jnp.ones((NSEG, NSEG), f32),
                 preferred_element_type=f32, precision=lax.Precision.HIGHEST)
    M = ((ur < uc) | ((ur == uc) & (jj16 < ii16))).astype(f32)
    rank = jnp.sum(M, axis=1, keepdims=True)
    S = (rank == jj16.astype(f32)).astype(f32)
    lo = jnp.sum(S * uc, axis=0, keepdims=True)
    Q = (ii16 == jj16 + 1).astype(f32)
    hi = jnp.dot(lo, Q, preferred_element_type=f32, precision=lax.Precision.HIGHEST)
    hi = jnp.minimum(jnp.where(lane == NSEG - 1, float(N), hi), float(N))
    seg_t = jnp.minimum(jnp.floor(lo * (1.0 / TLE)), float(NT2 - 1))

    o8r = lax.broadcasted_iota(i32, (E, E), 0)
    o8c = lax.broadcasted_iota(i32, (E, E), 1)
    D8 = jnp.where(o8r == o8c, jnp.broadcast_to(offs, (E, E)), 0.0)
    offs_col = jnp.dot(D8, jnp.ones((E, NSEG), f32),
                       preferred_element_type=f32, precision=lax.Precision.HIGHEST)
    lo_row = jnp.broadcast_to(lo, (E, NSEG))
    erow = lax.broadcasted_iota(i32, (E, NSEG), 0)
    seg_e = jnp.sum(((offs_col <= lo_row) & (erow >= 1)).astype(f32),
                    axis=0, keepdims=True)

    st_ref[...] = seg_t.astype(i32)
    se_ref[...] = seg_e.astype(i32)
    slo_ref[...] = lo.astype(i32)
    shi_ref[...] = hi.astype(i32)


def _enc_route(ids2d, x, W1, b1, W2, b2):
    seg = jax.ShapeDtypeStruct((1, NSEG), jnp.int32)
    return pl.pallas_call(
        _enc_route_body,
        grid=(N // TILE,),
        in_specs=[
            pl.BlockSpec((N, 1), lambda i: (0, 0)),
            pl.BlockSpec((TILE, SEQ), lambda i: (i, 0)),
            pl.BlockSpec((SEQ, HID), lambda i: (0, 0)),
            pl.BlockSpec((1, HID), lambda i: (0, 0)),
            pl.BlockSpec((HID, ENC), lambda i: (0, 0)),
            pl.BlockSpec((1, ENC), lambda i: (0, 0)),
        ],
        out_specs=[
            pl.BlockSpec((TILE, ENC), lambda i: (i, 0)),
            pl.BlockSpec((TILE, 1), lambda i: (i, 0)),
            pl.BlockSpec((1, NSEG), lambda i: (0, 0)),
            pl.BlockSpec((1, NSEG), lambda i: (0, 0)),
            pl.BlockSpec((1, NSEG), lambda i: (0, 0)),
            pl.BlockSpec((1, NSEG), lambda i: (0, 0)),
        ],
        out_shape=[
            jax.ShapeDtypeStruct((N, ENC), jnp.float32),
            jax.ShapeDtypeStruct((N, 1), jnp.int32),
            seg, seg, seg, seg,
        ],
    )(ids2d, x.astype(jnp.bfloat16), W1.astype(jnp.bfloat16),
      b1.reshape(1, HID), W2.astype(jnp.bfloat16), b2.reshape(1, ENC))


def _scatter_body(h_hbm, pos_hbm, hs_hbm, idx_v, rows_v, sem):
    wid = lax.axis_index("s") * 2 + lax.axis_index("c")
    base = wid * RPW
    pltpu.sync_copy(pos_hbm.at[pl.ds(base, RPW)], idx_v)
    pltpu.sync_copy(h_hbm.at[pl.ds(base, RPW)], rows_v)
    pltpu.async_copy(rows_v, hs_hbm.at[idx_v], sem).wait()


def _gather_body(outs_hbm, pos_hbm, out_hbm, idx_v, rows_v, sem):
    wid = lax.axis_index("s") * 2 + lax.axis_index("c")
    base = wid * RPW
    for c in range(RPW // GCH):
        pltpu.sync_copy(pos_hbm.at[pl.ds(base + c * GCH, GCH)], idx_v)
        pltpu.async_copy(outs_hbm.at[idx_v], rows_v, sem).wait()
        pltpu.sync_copy(rows_v, out_hbm.at[pl.ds(base + c * GCH, GCH)])


def _sc_kernels():
    global _SC_KERNELS
    if _SC_KERNELS is None:
        mesh = plsc.VectorSubcoreMesh(core_axis_name="c", subcore_axis_name="s")
        scatter_h = functools.partial(
            pl.kernel,
            out_type=jax.ShapeDtypeStruct((N, ENC), jnp.float32),
            mesh=mesh,
            scratch_types=[
                pltpu.VMEM((RPW,), jnp.int32),
                pltpu.VMEM((RPW, ENC), jnp.float32),
                pltpu.SemaphoreType.DMA,
            ],
        )(_scatter_body)
        gather_out = functools.partial(
            pl.kernel,
            out_type=jax.ShapeDtypeStruct((N, SEQ), jnp.float32),
            mesh=mesh,
            scratch_types=[
                pltpu.VMEM((GCH,), jnp.int32),
                pltpu.VMEM((GCH, SEQ), jnp.float32),
                pltpu.SemaphoreType.DMA,
            ],
        )(_gather_body)
        _SC_KERNELS = (scatter_h, gather_out)
    return _SC_KERNELS


def _expert_body(t_s, e_s, lo_s, hi_s, hs_ref, wv1_ref, bv1_ref, wv2_ref,
                 bv2_ref, out_ref):
    w = pl.program_id(0)
    t = t_s[w]
    lo = lo_s[w]
    hi = hi_s[w]
    h = hs_ref[...].astype(jnp.bfloat16)
    he = jnp.maximum(
        jnp.dot(h, wv1_ref[0], preferred_element_type=jnp.float32) + bv1_ref[0], 0.0)
    heb = he.astype(jnp.bfloat16)
    ye = jnp.dot(heb, wv2_ref[0], preferred_element_type=jnp.float32) + bv2_ref[0]
    gi = t * TLE + lax.broadcasted_iota(jnp.int32, (TLE, 1), 0)
    contrib = jnp.where((gi >= lo) & (gi < hi), ye, 0.0)
    first = jnp.logical_or(w == 0, t != t_s[jnp.maximum(w - 1, 0)])

    @pl.when(first)
    def _():
        out_ref[...] = contrib

    @pl.when(jnp.logical_not(first))
    def _():
        out_ref[...] += contrib


def _expert_heads(seg_t, seg_e, seg_lo, seg_hi, h_s, Wv1, bv1, Wv2, bv2):
    grid_spec = pltpu.PrefetchScalarGridSpec(
        num_scalar_prefetch=4,
        grid=(NITEMS,),
        in_specs=[
            pl.BlockSpec((TLE, ENC), lambda w, t, e, lo, hi: (t[w], 0)),
            pl.BlockSpec((1, ENC, HID), lambda w, t, e, lo, hi: (e[w], 0, 0)),
            pl.BlockSpec((1, 1, HID), lambda w, t, e, lo, hi: (e[w], 0, 0)),
            pl.BlockSpec((1, HID, SEQ), lambda w, t, e, lo, hi: (e[w], 0, 0)),
            pl.BlockSpec((1, 1, SEQ), lambda w, t, e, lo, hi: (e[w], 0, 0)),
        ],
        out_specs=pl.BlockSpec((TLE, SEQ), lambda w, t, e, lo, hi: (t[w], 0)),
    )
    return pl.pallas_call(
        _expert_body,
        grid_spec=grid_spec,
        out_shape=jax.ShapeDtypeStruct((N, SEQ), jnp.float32),
    )(seg_t, seg_e, seg_lo, seg_hi, h_s, Wv1.astype(jnp.bfloat16),
      bv1.reshape(E, 1, HID), Wv2.astype(jnp.bfloat16), bv2.reshape(E, 1, SEQ))


def kernel(x, W1, b1, W2, b2, Wv1, bv1, Wv2, bv2):
    ids2d = x[:, SEQ:SEQ + 1].astype(jnp.int32)
    scatter_h, gather_out = _sc_kernels()
    h, pos2d, st, se, slo, shi = _enc_route(ids2d, x, W1, b1, W2, b2)
    pos = pos2d.reshape(N)
    h_s = scatter_h(h, pos)
    out_s = _expert_heads(st.reshape(NSEG), se.reshape(NSEG), slo.reshape(NSEG),
                          shi.reshape(NSEG), h_s, Wv1, bv1, Wv2, bv2)
    return gather_out(out_s, pos)


# speedup vs baseline: 1.2401x; 1.2401x over previous
import functools

import jax
import jax.numpy as jnp
from jax import lax
from jax.experimental import pallas as pl
from jax.experimental.pallas import tpu as pltpu
from jax.experimental.pallas import tpu_sc as plsc

N = 4096
SEQ = 2048
HID = 1024
ENC = 512
E = 8
TILE = 512
TLE = 512
NT2 = N // TLE
NITEMS = NT2 + E - 1
NSEG = 32
NW = 32
RPW = N // NW
GCH = 32

_SC_KERNELS = None


def _enc_route_body(ids_ref, x_ref, w1_ref, b1_ref, w2_ref, b2_ref,
                    h_ref, pos_ref, st_ref, se_ref, slo_ref, shi_ref):
    t = pl.program_id(0)
    f32 = jnp.float32
    i32 = jnp.int32

    f = x_ref[...]
    h1 = jnp.maximum(
        jnp.dot(f, w1_ref[...], preferred_element_type=f32) + b1_ref[...], 0.0)
    h_ref[...] = jnp.maximum(
        jnp.dot(h1, w2_ref[...], preferred_element_type=f32) + b2_ref[...], 0.0)

    ids_full = ids_ref[...]
    onehot = (ids_full == lax.broadcasted_iota(i32, (N, E), 1)).astype(f32)
    counts = jnp.sum(onehot, axis=0, keepdims=True)
    tri = (lax.broadcasted_iota(i32, (E, E), 0)
           < lax.broadcasted_iota(i32, (E, E), 1)).astype(f32)
    c_hi = jnp.floor(counts * (1.0 / 256.0))
    c_lo = counts - 256.0 * c_hi
    offs = (jnp.dot(c_hi, tri, preferred_element_type=f32) * 256.0
            + jnp.dot(c_lo, tri, preferred_element_type=f32))

    r0 = t * TILE
    ids_tile = ids_ref[pl.ds(r0, TILE), :]
    onehot_t = (ids_tile == lax.broadcasted_iota(i32, (TILE, E), 1)).astype(f32)
    base_mask = (lax.broadcasted_iota(i32, (1, N), 1) < r0).astype(f32)
    base = jnp.dot(base_mask, onehot, preferred_element_type=f32)
    tril = (lax.broadcasted_iota(i32, (TILE, TILE), 1)
            <= lax.broadcasted_iota(i32, (TILE, TILE), 0)).astype(f32)
    cum = base + jnp.dot(tril, onehot_t, preferred_element_type=f32)
    posf = jnp.sum(onehot_t * (offs + cum - 1.0), axis=1, keepdims=True)
    pos_ref[...] = posf.astype(i32)

    @pl.when(t == N // TILE - 1)
    def _segments():
        lane = lax.broadcasted_iota(i32, (1, NSEG), 1)
        prow = lax.broadcasted_iota(i32, (E, NSEG), 0)
        pcol = lax.broadcasted_iota(i32, (E, NSEG), 1)
        P = ((pcol == prow + (NT2 - 1)) & (prow >= 1)).astype(f32)
        u_off = jnp.dot(offs, P, preferred_element_type=f32, precision=lax.Precision.HIGHEST)
        u = jnp.where(lane < NT2, lane.astype(f32) * TLE,
                      jnp.where(lane < NITEMS, u_off, float(N)))

        ii16 = lax.broadcasted_iota(i32, (NSEG, NSEG), 0)
        jj16 = lax.broadcasted_iota(i32, (NSEG, NSEG), 1)
        ur = jnp.broadcast_to(u, (NSEG, NSEG))
        D = jnp.where(ii16 == jj16, ur, 0.0)
        uc = jnp.dot(D, jnp.ones((NSEG, NSEG), f32),
                     preferred_element_type=f32, precision=lax.Precision.HIGHEST)
        M = ((ur < uc) | ((ur == uc) & (jj16 < ii16))).astype(f32)
        rank = jnp.sum(M, axis=1, keepdims=True)
        S = (rank == jj16.astype(f32)).astype(f32)
        lo = jnp.sum(S * uc, axis=0, keepdims=True)
        Q = (ii16 == jj16 + 1).astype(f32)
        hi = jnp.dot(lo, Q, preferred_element_type=f32, precision=lax.Precision.HIGHEST)
        hi = jnp.minimum(jnp.where(lane == NSEG - 1, float(N), hi), float(N))
        seg_t = jnp.minimum(jnp.floor(lo * (1.0 / TLE)), float(NT2 - 1))

        o8r = lax.broadcasted_iota(i32, (E, E), 0)
        o8c = lax.broadcasted_iota(i32, (E, E), 1)
        D8 = jnp.where(o8r == o8c, jnp.broadcast_to(offs, (E, E)), 0.0)
        offs_col = jnp.dot(D8, jnp.ones((E, NSEG), f32),
                           preferred_element_type=f32, precision=lax.Precision.HIGHEST)
        lo_row = jnp.broadcast_to(lo, (E, NSEG))
        erow = lax.broadcasted_iota(i32, (E, NSEG), 0)
        seg_e = jnp.sum(((offs_col <= lo_row) & (erow >= 1)).astype(f32),
                        axis=0, keepdims=True)

        st_ref[...] = seg_t.astype(i32)
        se_ref[...] = seg_e.astype(i32)
        slo_ref[...] = lo.astype(i32)
        shi_ref[...] = hi.astype(i32)


def _enc_route(ids2d, x, W1, b1, W2, b2):
    seg = jax.ShapeDtypeStruct((1, NSEG), jnp.int32)
    return pl.pallas_call(
        _enc_route_body,
        grid=(N // TILE,),
        in_specs=[
            pl.BlockSpec((N, 1), lambda i: (0, 0)),
            pl.BlockSpec((TILE, SEQ), lambda i: (i, 0)),
            pl.BlockSpec((SEQ, HID), lambda i: (0, 0)),
            pl.BlockSpec((1, HID), lambda i: (0, 0)),
            pl.BlockSpec((HID, ENC), lambda i: (0, 0)),
            pl.BlockSpec((1, ENC), lambda i: (0, 0)),
        ],
        out_specs=[
            pl.BlockSpec((TILE, ENC), lambda i: (i, 0)),
            pl.BlockSpec((TILE, 1), lambda i: (i, 0)),
            pl.BlockSpec((1, NSEG), lambda i: (0, 0)),
            pl.BlockSpec((1, NSEG), lambda i: (0, 0)),
            pl.BlockSpec((1, NSEG), lambda i: (0, 0)),
            pl.BlockSpec((1, NSEG), lambda i: (0, 0)),
        ],
        out_shape=[
            jax.ShapeDtypeStruct((N, ENC), jnp.float32),
            jax.ShapeDtypeStruct((N, 1), jnp.int32),
            seg, seg, seg, seg,
        ],
    )(ids2d, x, W1, b1.reshape(1, HID), W2, b2.reshape(1, ENC))


def _scatter_body(h_hbm, pos_hbm, hs_hbm, idx_v, rows_v, sem):
    wid = lax.axis_index("s") * 2 + lax.axis_index("c")
    base = wid * RPW
    pltpu.sync_copy(pos_hbm.at[pl.ds(base, RPW)], idx_v)
    pltpu.sync_copy(h_hbm.at[pl.ds(base, RPW)], rows_v)
    pltpu.async_copy(rows_v, hs_hbm.at[idx_v], sem).wait()


def _gather_body(outs_hbm, pos_hbm, out_hbm, idx_v, rows_v, sem):
    wid = lax.axis_index("s") * 2 + lax.axis_index("c")
    base = wid * RPW
    for c in range(RPW // GCH):
        pltpu.sync_copy(pos_hbm.at[pl.ds(base + c * GCH, GCH)], idx_v)
        pltpu.async_copy(outs_hbm.at[idx_v], rows_v, sem).wait()
        pltpu.sync_copy(rows_v, out_hbm.at[pl.ds(base + c * GCH, GCH)])


def _sc_kernels():
    global _SC_KERNELS
    if _SC_KERNELS is None:
        mesh = plsc.VectorSubcoreMesh(core_axis_name="c", subcore_axis_name="s")
        scatter_h = functools.partial(
            pl.kernel,
            out_type=jax.ShapeDtypeStruct((N, ENC), jnp.float32),
            mesh=mesh,
            scratch_types=[
                pltpu.VMEM((RPW,), jnp.int32),
                pltpu.VMEM((RPW, ENC), jnp.float32),
                pltpu.SemaphoreType.DMA,
            ],
        )(_scatter_body)
        gather_out = functools.partial(
            pl.kernel,
            out_type=jax.ShapeDtypeStruct((N, SEQ), jnp.float32),
            mesh=mesh,
            scratch_types=[
                pltpu.VMEM((GCH,), jnp.int32),
                pltpu.VMEM((GCH, SEQ), jnp.float32),
                pltpu.SemaphoreType.DMA,
            ],
        )(_gather_body)
        _SC_KERNELS = (scatter_h, gather_out)
    return _SC_KERNELS


def _expert_body(t_s, e_s, lo_s, hi_s, hs_ref, wv1_ref, bv1_ref, wv2_ref,
                 bv2_ref, out_ref):
    w = pl.program_id(0)
    t = t_s[w]
    lo = lo_s[w]
    hi = hi_s[w]
    h = hs_ref[...]
    he = jnp.maximum(
        jnp.dot(h, wv1_ref[0], preferred_element_type=jnp.float32) + bv1_ref[0], 0.0)
    ye = jnp.dot(he, wv2_ref[0], preferred_element_type=jnp.float32) + bv2_ref[0]
    gi = t * TLE + lax.broadcasted_iota(jnp.int32, (TLE, 1), 0)
    contrib = jnp.where((gi >= lo) & (gi < hi), ye, 0.0)
    first = jnp.logical_or(w == 0, t != t_s[jnp.maximum(w - 1, 0)])

    @pl.when(first)
    def _():
        out_ref[...] = contrib

    @pl.when(jnp.logical_not(first))
    def _():
        out_ref[...] += contrib


def _expert_heads(seg_t, seg_e, seg_lo, seg_hi, h_s, Wv1, bv1, Wv2, bv2):
    grid_spec = pltpu.PrefetchScalarGridSpec(
        num_scalar_prefetch=4,
        grid=(NITEMS,),
        in_specs=[
            pl.BlockSpec((TLE, ENC), lambda w, t, e, lo, hi: (t[w], 0)),
            pl.BlockSpec((1, ENC, HID), lambda w, t, e, lo, hi: (e[w], 0, 0)),
            pl.BlockSpec((1, 1, HID), lambda w, t, e, lo, hi: (e[w], 0, 0)),
            pl.BlockSpec((1, HID, SEQ), lambda w, t, e, lo, hi: (e[w], 0, 0)),
            pl.BlockSpec((1, 1, SEQ), lambda w, t, e, lo, hi: (e[w], 0, 0)),
        ],
        out_specs=pl.BlockSpec((TLE, SEQ), lambda w, t, e, lo, hi: (t[w], 0)),
    )
    return pl.pallas_call(
        _expert_body,
        grid_spec=grid_spec,
        out_shape=jax.ShapeDtypeStruct((N, SEQ), jnp.float32),
    )(seg_t, seg_e, seg_lo, seg_hi, h_s, Wv1, bv1.reshape(E, 1, HID), Wv2,
      bv2.reshape(E, 1, SEQ))


def kernel(x, W1, b1, W2, b2, Wv1, bv1, Wv2, bv2):
    ids2d = x[:, SEQ:SEQ + 1].astype(jnp.int32)
    scatter_h, gather_out = _sc_kernels()
    h, pos2d, st, se, slo, shi = _enc_route(ids2d, x, W1, b1, W2, b2)
    pos = pos2d.reshape(N)
    h_s = scatter_h(h, pos)
    out_s = _expert_heads(st.reshape(NSEG), se.reshape(NSEG), slo.reshape(NSEG),
                          shi.reshape(NSEG), h_s, Wv1, bv1, Wv2, bv2)
    return gather_out(out_s, pos)


# speedup vs baseline: 1.2486x; 1.0068x over previous
import functools

import jax
import jax.numpy as jnp
from jax import lax
from jax.experimental import pallas as pl
from jax.experimental.pallas import tpu as pltpu
from jax.experimental.pallas import tpu_sc as plsc

N = 4096
SEQ = 2048
HID = 1024
ENC = 512
E = 8
TILE = 512
TLE = 512
NT2 = N // TLE
NITEMS = NT2 + E - 1
NSEG = 32
NW = 32
RPW = N // NW
GCH = 16

_SC_KERNELS = None


def _enc_route_body(ids_ref, x_ref, w1_ref, b1_ref, w2_ref, b2_ref,
                    h_ref, pos_ref, st_ref, se_ref, slo_ref, shi_ref):
    t = pl.program_id(0)
    f32 = jnp.float32
    i32 = jnp.int32

    f = x_ref[...]
    h1 = jnp.maximum(
        jnp.dot(f, w1_ref[...], preferred_element_type=f32) + b1_ref[...], 0.0)
    h_ref[...] = jnp.maximum(
        jnp.dot(h1, w2_ref[...], preferred_element_type=f32) + b2_ref[...], 0.0)

    ids_full = ids_ref[...]
    onehot = (ids_full == lax.broadcasted_iota(i32, (N, E), 1)).astype(f32)
    counts = jnp.sum(onehot, axis=0, keepdims=True)
    tri = (lax.broadcasted_iota(i32, (E, E), 0)
           < lax.broadcasted_iota(i32, (E, E), 1)).astype(f32)
    c_hi = jnp.floor(counts * (1.0 / 256.0))
    c_lo = counts - 256.0 * c_hi
    offs = (jnp.dot(c_hi, tri, preferred_element_type=f32) * 256.0
            + jnp.dot(c_lo, tri, preferred_element_type=f32))

    r0 = t * TILE
    ids_tile = ids_ref[pl.ds(r0, TILE), :]
    onehot_t = (ids_tile == lax.broadcasted_iota(i32, (TILE, E), 1)).astype(f32)
    base_mask = (lax.broadcasted_iota(i32, (1, N), 1) < r0).astype(f32)
    base = jnp.dot(base_mask, onehot, preferred_element_type=f32)
    tril = (lax.broadcasted_iota(i32, (TILE, TILE), 1)
            <= lax.broadcasted_iota(i32, (TILE, TILE), 0)).astype(f32)
    cum = base + jnp.dot(tril, onehot_t, preferred_element_type=f32)
    posf = jnp.sum(onehot_t * (offs + cum - 1.0), axis=1, keepdims=True)
    pos_ref[...] = posf.astype(i32)

    @pl.when(t == N // TILE - 1)
    def _segments():
        lane = lax.broadcasted_iota(i32, (1, NSEG), 1)
        prow = lax.broadcasted_iota(i32, (E, NSEG), 0)
        pcol = lax.broadcasted_iota(i32, (E, NSEG), 1)
        P = ((pcol == prow + (NT2 - 1)) & (prow >= 1)).astype(f32)
        u_off = jnp.dot(offs, P, preferred_element_type=f32, precision=lax.Precision.HIGHEST)
        u = jnp.where(lane < NT2, lane.astype(f32) * TLE,
                      jnp.where(lane < NITEMS, u_off, float(N)))

        ii16 = lax.broadcasted_iota(i32, (NSEG, NSEG), 0)
        jj16 = lax.broadcasted_iota(i32, (NSEG, NSEG), 1)
        ur = jnp.broadcast_to(u, (NSEG, NSEG))
        D = jnp.where(ii16 == jj16, ur, 0.0)
        uc = jnp.dot(D, jnp.ones((NSEG, NSEG), f32),
                     preferred_element_type=f32, precision=lax.Precision.HIGHEST)
        M = ((ur < uc) | ((ur == uc) & (jj16 < ii16))).astype(f32)
        rank = jnp.sum(M, axis=1, keepdims=True)
        S = (rank == jj16.astype(f32)).astype(f32)
        lo = jnp.sum(S * uc, axis=0, keepdims=True)
        Q = (ii16 == jj16 + 1).astype(f32)
        hi = jnp.dot(lo, Q, preferred_element_type=f32, precision=lax.Precision.HIGHEST)
        hi = jnp.minimum(jnp.where(lane == NSEG - 1, float(N), hi), float(N))
        seg_t = jnp.minimum(jnp.floor(lo * (1.0 / TLE)), float(NT2 - 1))

        o8r = lax.broadcasted_iota(i32, (E, E), 0)
        o8c = lax.broadcasted_iota(i32, (E, E), 1)
        D8 = jnp.where(o8r == o8c, jnp.broadcast_to(offs, (E, E)), 0.0)
        offs_col = jnp.dot(D8, jnp.ones((E, NSEG), f32),
                           preferred_element_type=f32, precision=lax.Precision.HIGHEST)
        lo_row = jnp.broadcast_to(lo, (E, NSEG))
        erow = lax.broadcasted_iota(i32, (E, NSEG), 0)
        seg_e = jnp.sum(((offs_col <= lo_row) & (erow >= 1)).astype(f32),
                        axis=0, keepdims=True)

        st_ref[...] = seg_t.astype(i32)
        se_ref[...] = seg_e.astype(i32)
        slo_ref[...] = lo.astype(i32)
        shi_ref[...] = hi.astype(i32)


def _enc_route(ids2d, x, W1, b1, W2, b2):
    seg = jax.ShapeDtypeStruct((1, NSEG), jnp.int32)
    return pl.pallas_call(
        _enc_route_body,
        grid=(N // TILE,),
        in_specs=[
            pl.BlockSpec((N, 1), lambda i: (0, 0)),
            pl.BlockSpec((TILE, SEQ), lambda i: (i, 0)),
            pl.BlockSpec((SEQ, HID), lambda i: (0, 0)),
            pl.BlockSpec((1, HID), lambda i: (0, 0)),
            pl.BlockSpec((HID, ENC), lambda i: (0, 0)),
            pl.BlockSpec((1, ENC), lambda i: (0, 0)),
        ],
        out_specs=[
            pl.BlockSpec((TILE, ENC), lambda i: (i, 0)),
            pl.BlockSpec((TILE, 1), lambda i: (i, 0)),
            pl.BlockSpec((1, NSEG), lambda i: (0, 0)),
            pl.BlockSpec((1, NSEG), lambda i: (0, 0)),
            pl.BlockSpec((1, NSEG), lambda i: (0, 0)),
            pl.BlockSpec((1, NSEG), lambda i: (0, 0)),
        ],
        out_shape=[
            jax.ShapeDtypeStruct((N, ENC), jnp.float32),
            jax.ShapeDtypeStruct((N, 1), jnp.int32),
            seg, seg, seg, seg,
        ],
    )(ids2d, x, W1, b1.reshape(1, HID), W2, b2.reshape(1, ENC))


def _scatter_body(h_hbm, pos_hbm, hs_hbm, idx_v, rows_v, sem):
    wid = lax.axis_index("s") * 2 + lax.axis_index("c")
    base = wid * RPW
    pltpu.sync_copy(pos_hbm.at[pl.ds(base, RPW)], idx_v)
    pltpu.sync_copy(h_hbm.at[pl.ds(base, RPW)], rows_v)
    pltpu.async_copy(rows_v, hs_hbm.at[idx_v], sem).wait()


def _gather_body(outs_hbm, pos_hbm, out_hbm, idx_a, idx_b, rows_a, rows_b,
                 sem_a, sem_b):
    wid = lax.axis_index("s") * 2 + lax.axis_index("c")
    base = wid * RPW
    nch = RPW // GCH
    idx = (idx_a, idx_b)
    rows = (rows_a, rows_b)
    sems = (sem_a, sem_b)
    copies = []
    for c in range(nch):
        b = c % 2
        pltpu.sync_copy(pos_hbm.at[pl.ds(base + c * GCH, GCH)], idx[b])
        copies.append(pltpu.async_copy(outs_hbm.at[idx[b]], rows[b], sems[b]))
        if c > 0:
            copies[c - 1].wait()
            pltpu.sync_copy(rows[(c - 1) % 2],
                            out_hbm.at[pl.ds(base + (c - 1) * GCH, GCH)])
    copies[nch - 1].wait()
    pltpu.sync_copy(rows[(nch - 1) % 2],
                    out_hbm.at[pl.ds(base + (nch - 1) * GCH, GCH)])


def _sc_kernels():
    global _SC_KERNELS
    if _SC_KERNELS is None:
        mesh = plsc.VectorSubcoreMesh(core_axis_name="c", subcore_axis_name="s")
        scatter_h = functools.partial(
            pl.kernel,
            out_type=jax.ShapeDtypeStruct((N, ENC), jnp.float32),
            mesh=mesh,
            scratch_types=[
                pltpu.VMEM((RPW,), jnp.int32),
                pltpu.VMEM((RPW, ENC), jnp.float32),
                pltpu.SemaphoreType.DMA,
            ],
        )(_scatter_body)
        gather_out = functools.partial(
            pl.kernel,
            out_type=jax.ShapeDtypeStruct((N, SEQ), jnp.float32),
            mesh=mesh,
            scratch_types=[
                pltpu.VMEM((GCH,), jnp.int32),
                pltpu.VMEM((GCH,), jnp.int32),
                pltpu.VMEM((GCH, SEQ), jnp.float32),
                pltpu.VMEM((GCH, SEQ), jnp.float32),
                pltpu.SemaphoreType.DMA,
                pltpu.SemaphoreType.DMA,
            ],
        )(_gather_body)
        _SC_KERNELS = (scatter_h, gather_out)
    return _SC_KERNELS


def _expert_body(t_s, e_s, lo_s, hi_s, hs_ref, wv1_ref, bv1_ref, wv2_ref,
                 bv2_ref, out_ref):
    w = pl.program_id(0)
    t = t_s[w]
    lo = lo_s[w]
    hi = hi_s[w]
    h = hs_ref[...]
    he = jnp.maximum(
        jnp.dot(h, wv1_ref[0], preferred_element_type=jnp.float32) + bv1_ref[0], 0.0)
    ye = jnp.dot(he, wv2_ref[0], preferred_element_type=jnp.float32) + bv2_ref[0]
    gi = t * TLE + lax.broadcasted_iota(jnp.int32, (TLE, 1), 0)
    contrib = jnp.where((gi >= lo) & (gi < hi), ye, 0.0)
    first = jnp.logical_or(w == 0, t != t_s[jnp.maximum(w - 1, 0)])

    @pl.when(first)
    def _():
        out_ref[...] = contrib

    @pl.when(jnp.logical_not(first))
    def _():
        out_ref[...] += contrib


def _expert_heads(seg_t, seg_e, seg_lo, seg_hi, h_s, Wv1, bv1, Wv2, bv2):
    grid_spec = pltpu.PrefetchScalarGridSpec(
        num_scalar_prefetch=4,
        grid=(NITEMS,),
        in_specs=[
            pl.BlockSpec((TLE, ENC), lambda w, t, e, lo, hi: (t[w], 0)),
            pl.BlockSpec((1, ENC, HID), lambda w, t, e, lo, hi: (e[w], 0, 0)),
            pl.BlockSpec((1, 1, HID), lambda w, t, e, lo, hi: (e[w], 0, 0)),
            pl.BlockSpec((1, HID, SEQ), lambda w, t, e, lo, hi: (e[w], 0, 0)),
            pl.BlockSpec((1, 1, SEQ), lambda w, t, e, lo, hi: (e[w], 0, 0)),
        ],
        out_specs=pl.BlockSpec((TLE, SEQ), lambda w, t, e, lo, hi: (t[w], 0)),
    )
    return pl.pallas_call(
        _expert_body,
        grid_spec=grid_spec,
        out_shape=jax.ShapeDtypeStruct((N, SEQ), jnp.float32),
    )(seg_t, seg_e, seg_lo, seg_hi, h_s, Wv1, bv1.reshape(E, 1, HID), Wv2,
      bv2.reshape(E, 1, SEQ))


def kernel(x, W1, b1, W2, b2, Wv1, bv1, Wv2, bv2):
    ids2d = x[:, SEQ:SEQ + 1].astype(jnp.int32)
    scatter_h, gather_out = _sc_kernels()
    h, pos2d, st, se, slo, shi = _enc_route(ids2d, x, W1, b1, W2, b2)
    pos = pos2d.reshape(N)
    h_s = scatter_h(h, pos)
    out_s = _expert_heads(st.reshape(NSEG), se.reshape(NSEG), slo.reshape(NSEG),
                          shi.reshape(NSEG), h_s, Wv1, bv1, Wv2, bv2)
    return gather_out(out_s, pos)
